# baseline (speedup 1.0000x reference)
import functools

import jax
import jax.numpy as jnp
from jax import lax
from jax.experimental import pallas as pl
from jax.experimental.pallas import tpu as pltpu
from jax.experimental.pallas import tpu_sc as plsc

N = 10000
E = 320000
D = 128
DH = D // 2

NC = 2
NS = 16
CHUNK = 64
NBUF = 2
SEG = 20
NSEG = 16

CHUNKS_PER_TILE = SEG * NSEG
E_PAD = NS * CHUNKS_PER_TILE * CHUNK
N_PAD = 10112
ROWS_PER_TILE = N_PAD // NS
TROWS = N // NS


def _make_sc_agg(width, with_counts):
    def body(xh_hbm, src_hbm, dst_hbm, *rest):
        if with_counts:
            (agg_out, cnt_out, src_seg, dst_seg, rows_v, ones_v, zcnt_v,
             table_sh, agg_sh, cnt_sh, *sems) = rest
        else:
            (agg_out, src_seg, dst_seg, rows_v,
             table_sh, agg_sh, *sems) = rest
            cnt_out = cnt_sh = ones_v = zcnt_v = None
        gsems = sems[0:NBUF]
        ssems = sems[NBUF:2 * NBUF]
        isems = sems[2 * NBUF:2 * NBUF + 2]
        csem = sems[-1]

        c = lax.axis_index("c")
        s = lax.axis_index("s")
        zero16 = jnp.zeros((16,), jnp.float32)
        nv = width // 16

        def _zrow(i, carry):
            rows_v[0, i // nv, pl.ds((i % nv) * 16, 16)] = zero16
            return carry
        lax.fori_loop(0, CHUNK * nv, _zrow, 0)

        base = pl.multiple_of(s * ROWS_PER_TILE, 8)
        for k in range(ROWS_PER_TILE // CHUNK):
            pltpu.sync_copy(rows_v.at[0],
                            agg_sh.at[pl.ds(base + k * CHUNK, CHUNK)])
        tail = ROWS_PER_TILE % CHUNK
        if tail:
            pltpu.sync_copy(
                rows_v.at[0, pl.ds(0, tail)],
                agg_sh.at[pl.ds(base + ROWS_PER_TILE - tail, tail)])

        if with_counts:
            ones16 = jnp.ones((16,), jnp.float32)

            def _ones(i, carry):
                ones_v[i] = ones16
                zcnt_v[i] = zero16
                return carry
            lax.fori_loop(0, CHUNK, _ones, 0)
            for k in range(ROWS_PER_TILE // CHUNK):
                pltpu.sync_copy(zcnt_v,
                                cnt_sh.at[pl.ds(base + k * CHUNK, CHUNK)])
            if tail:
                pltpu.sync_copy(
                    zcnt_v.at[pl.ds(0, tail)],
                    cnt_sh.at[pl.ds(base + ROWS_PER_TILE - tail, tail)])

        trow = s * TROWS
        pltpu.sync_copy(xh_hbm.at[c, pl.ds(trow, TROWS)],
                        table_sh.at[pl.ds(trow, TROWS)])

        def _load_seg(seg, sp):
            r = s * CHUNKS_PER_TILE + seg * SEG
            pltpu.async_copy(src_hbm.at[pl.ds(r, SEG)], src_seg.at[sp],
                             isems[sp])
            pltpu.async_copy(dst_hbm.at[pl.ds(r, SEG)], dst_seg.at[sp],
                             isems[sp])

        def _wait_seg(seg, sp):
            r = s * CHUNKS_PER_TILE + seg * SEG
            pltpu.make_async_copy(src_hbm.at[pl.ds(r, SEG)], src_seg.at[sp],
                                  isems[sp]).wait()
            pltpu.make_async_copy(dst_hbm.at[pl.ds(r, SEG)], dst_seg.at[sp],
                                  isems[sp]).wait()

        _load_seg(0, 0)
        _load_seg(1, 1)

        plsc.subcore_barrier()

        def _src_ref(lc):
            if lc % 3 == 2:
                return xh_hbm.at[c]
            return table_sh

        def _gather(sp, lc, b):
            return pltpu.async_copy(_src_ref(lc).at[src_seg.at[sp, lc]],
                                    rows_v.at[b], gsems[b])

        def _wait_gather(sp, lc, b):
            pltpu.make_async_copy(_src_ref(lc).at[src_seg.at[sp, lc]],
                                  rows_v.at[b], gsems[b]).wait()

        def _scatter(sp, lc, b):
            return pltpu.async_copy(rows_v.at[b],
                                    agg_sh.at[dst_seg.at[sp, lc]],
                                    ssems[b], add=True)

        def _wait_scatter(sp, lc, b):
            pltpu.make_async_copy(rows_v.at[b],
                                  agg_sh.at[dst_seg.at[sp, lc]],
                                  ssems[b]).wait()

        def _run_segment(seg, sp):
            _wait_seg(seg, sp)
            _gather(sp, 0, 0)
            for lc in range(SEG):
                b = lc % NBUF
                nb = (lc + 1) % NBUF
                if lc + 1 < SEG:
                    if lc >= 1:
                        _wait_scatter(sp, lc - 1, nb)
                    _gather(sp, lc + 1, nb)
                _wait_gather(sp, lc, b)
                _scatter(sp, lc, b)
                if with_counts:
                    @pl.when(c == sp)
                    def _cnt():
                        pltpu.async_copy(ones_v,
                                         cnt_sh.at[dst_seg.at[sp, lc]],
                                         csem, add=True)
            _wait_scatter(sp, SEG - 2, (SEG - 2) % NBUF)
            _wait_scatter(sp, SEG - 1, (SEG - 1) % NBUF)

        def _pair(g, carry):
            for sp in range(2):
                seg = g * 2 + sp
                _run_segment(seg, sp)

                @pl.when(g < NSEG // 2 - 1)
                def _next():
                    _load_seg(seg + 2, sp)
            return carry
        lax.fori_loop(0, NSEG // 2, _pair, 0)

        if with_counts:
            def _cd(i, carry):
                pltpu.make_async_copy(ones_v, cnt_sh.at[dst_seg.at[0, 0]],
                                      csem).wait()
                return carry
            lax.fori_loop(0, CHUNKS_PER_TILE // 2, _cd, 0)

        plsc.subcore_barrier()

        pltpu.sync_copy(agg_sh.at[pl.ds(base, ROWS_PER_TILE)],
                        agg_out.at[c, pl.ds(base, ROWS_PER_TILE)])
        if with_counts:
            pltpu.sync_copy(cnt_sh.at[pl.ds(base, ROWS_PER_TILE)],
                            cnt_out.at[c, pl.ds(base, ROWS_PER_TILE)])

    mesh = plsc.VectorSubcoreMesh(core_axis_name="c", subcore_axis_name="s",
                                  num_cores=NC, num_subcores=NS)
    out_type = [jax.ShapeDtypeStruct((NC, N_PAD, width), jnp.float32)]
    scratch = [
        pltpu.VMEM((2, SEG, CHUNK), jnp.int32),
        pltpu.VMEM((2, SEG, CHUNK), jnp.int32),
        pltpu.VMEM((NBUF, CHUNK, width), jnp.float32),
    ]
    if with_counts:
        out_type.append(jax.ShapeDtypeStruct((NC, N_PAD, 16), jnp.float32))
        scratch.append(pltpu.VMEM((CHUNK, 16), jnp.float32))
        scratch.append(pltpu.VMEM((CHUNK, 16), jnp.float32))
    scratch.append(pltpu.VMEM_SHARED((N, width), jnp.float32))
    scratch.append(pltpu.VMEM_SHARED((N_PAD, width), jnp.float32))
    if with_counts:
        scratch.append(pltpu.VMEM_SHARED((N_PAD, 16), jnp.float32))
    scratch += [pltpu.SemaphoreType.DMA] * (2 * NBUF + 3)
    return pl.kernel(
        body, out_type=tuple(out_type), mesh=mesh,
        compiler_params=pltpu.CompilerParams(use_tc_tiling_on_sc=False),
        scratch_types=scratch)


_sc_agg1 = _make_sc_agg(DH, True)
_sc_agg2 = _make_sc_agg(DH, False)

BR = 2000


def _tc1_body(p_ref, cnt_ref, x_ref, wl_ref, wr_ref, b_ref, h_ref, hs_ref):
    cnt = jnp.maximum(cnt_ref[0, :, 0:1] + cnt_ref[1, :, 0:1], 1.0)
    mean = jnp.concatenate([p_ref[0], p_ref[1]], axis=1) / cnt
    h = (jnp.dot(mean, wl_ref[...], preferred_element_type=jnp.float32)
         + jnp.dot(x_ref[...], wr_ref[...], preferred_element_type=jnp.float32)
         + b_ref[...])
    h = jnp.maximum(h, 0.0)
    h_ref[...] = h
    hs_ref[0] = h[:, :DH]
    hs_ref[1] = h[:, DH:]


def _tc1(p, cnt, x, wl_t, wr_t, b):
    return pl.pallas_call(
        _tc1_body,
        grid=(N // BR,),
        in_specs=[
            pl.BlockSpec((NC, BR, DH), lambda i: (0, i, 0)),
            pl.BlockSpec((NC, BR, 16), lambda i: (0, i, 0)),
            pl.BlockSpec((BR, D), lambda i: (i, 0)),
            pl.BlockSpec((D, D), lambda i: (0, 0)),
            pl.BlockSpec((D, D), lambda i: (0, 0)),
            pl.BlockSpec((1, D), lambda i: (0, 0)),
        ],
        out_specs=[pl.BlockSpec((BR, D), lambda i: (i, 0)),
                   pl.BlockSpec((NC, BR, DH), lambda i: (0, i, 0))],
        out_shape=[jax.ShapeDtypeStruct((N, D), jnp.float32),
                   jax.ShapeDtypeStruct((NC, N, DH), jnp.float32)],
    )(p, cnt, x, wl_t, wr_t, b)


def _tc2_body(p_ref, cnt_ref, x_ref, wl_ref, wr_ref, b_ref, o_ref):
    cnt = jnp.maximum(cnt_ref[0, :, 0:1] + cnt_ref[1, :, 0:1], 1.0)
    mean = jnp.concatenate([p_ref[0], p_ref[1]], axis=1) / cnt
    o_ref[...] = (
        jnp.dot(mean, wl_ref[...], preferred_element_type=jnp.float32)
        + jnp.dot(x_ref[...], wr_ref[...], preferred_element_type=jnp.float32)
        + b_ref[...])


def _tc2(p, cnt, x, wl_t, wr_t, b):
    return pl.pallas_call(
        _tc2_body,
        grid=(N // BR,),
        in_specs=[
            pl.BlockSpec((NC, BR, DH), lambda i: (0, i, 0)),
            pl.BlockSpec((NC, BR, 16), lambda i: (0, i, 0)),
            pl.BlockSpec((BR, D), lambda i: (i, 0)),
            pl.BlockSpec((D, D), lambda i: (0, 0)),
            pl.BlockSpec((D, D), lambda i: (0, 0)),
            pl.BlockSpec((1, D), lambda i: (0, 0)),
        ],
        out_specs=pl.BlockSpec((BR, D), lambda i: (i, 0)),
        out_shape=jax.ShapeDtypeStruct((N, D), jnp.float32),
    )(p, cnt, x, wl_t, wr_t, b)


def kernel(x, edge_index, W1_l, b1_l, W1_r, W2_l, b2_l, W2_r):
    src = edge_index[0].astype(jnp.int32)
    dst = edge_index[1].astype(jnp.int32)
    pad = E_PAD - E
    src2d = jnp.concatenate(
        [src, jnp.zeros((pad,), jnp.int32)]).reshape(-1, CHUNK)
    dst2d = jnp.concatenate(
        [dst, jnp.full((pad,), N, jnp.int32)]).reshape(-1, CHUNK)

    xh = x.reshape(N, NC, DH).transpose(1, 0, 2)

    agg1, cnt = _sc_agg1(xh, src2d, dst2d)
    h, hs = _tc1(agg1, cnt, x, W1_l.T, W1_r.T, b1_l.reshape(1, D))
    (agg2,) = _sc_agg2(hs, src2d, dst2d)
    out = _tc2(agg2, cnt, h, W2_l.T, W2_r.T, b2_l.reshape(1, D))
    return out


# speedup vs baseline: 1.2954x; 1.2954x over previous
import functools

import jax
import jax.numpy as jnp
from jax import lax
from jax.experimental import pallas as pl
from jax.experimental.pallas import tpu as pltpu
from jax.experimental.pallas import tpu_sc as plsc

N = 10000
E = 320000
D = 128
DH = D // 2

NC = 2
NS = 16
CHUNK = 64
NBUF = 2
SEG = 20
NSEG = 16

CHUNKS_PER_TILE = SEG * NSEG
E_PAD = NS * CHUNKS_PER_TILE * CHUNK
N_PAD = 10112
ROWS_PER_TILE = N_PAD // NS
TROWS = N // NS


def _make_sc_agg(width, with_counts):
    def body(xh_hbm, src_hbm, dst_hbm, *rest):
        if with_counts:
            (agg_out, cnt_out, src_seg, dst_seg, rows_v, ones_v, zcnt_v,
             table_sh, agg_sh, cnt_sh, *sems) = rest
        else:
            (agg_out, src_seg, dst_seg, rows_v,
             table_sh, agg_sh, *sems) = rest
            cnt_out = cnt_sh = ones_v = zcnt_v = None
        gsems = sems[0:NBUF]
        ssems = sems[NBUF:2 * NBUF]
        isems = sems[2 * NBUF:2 * NBUF + 2]
        csem = sems[-1]

        c = lax.axis_index("c")
        s = lax.axis_index("s")
        zero16 = jnp.zeros((16,), jnp.float32)
        nv = width // 16

        def _zrow(i, carry):
            rows_v[0, i // nv, pl.ds((i % nv) * 16, 16)] = zero16
            return carry
        lax.fori_loop(0, CHUNK * nv, _zrow, 0)

        base = pl.multiple_of(s * ROWS_PER_TILE, 8)
        for k in range(ROWS_PER_TILE // CHUNK):
            pltpu.sync_copy(rows_v.at[0],
                            agg_sh.at[pl.ds(base + k * CHUNK, CHUNK)])
        tail = ROWS_PER_TILE % CHUNK
        if tail:
            pltpu.sync_copy(
                rows_v.at[0, pl.ds(0, tail)],
                agg_sh.at[pl.ds(base + ROWS_PER_TILE - tail, tail)])

        if with_counts:
            ones16 = jnp.ones((16,), jnp.float32)

            def _ones(i, carry):
                ones_v[i] = ones16
                zcnt_v[i] = zero16
                return carry
            lax.fori_loop(0, CHUNK, _ones, 0)
            for k in range(ROWS_PER_TILE // CHUNK):
                pltpu.sync_copy(zcnt_v,
                                cnt_sh.at[pl.ds(base + k * CHUNK, CHUNK)])
            if tail:
                pltpu.sync_copy(
                    zcnt_v.at[pl.ds(0, tail)],
                    cnt_sh.at[pl.ds(base + ROWS_PER_TILE - tail, tail)])

        trow = s * TROWS
        pltpu.sync_copy(xh_hbm.at[c, pl.ds(trow, TROWS)],
                        table_sh.at[pl.ds(trow, TROWS)])

        def _load_seg(seg, sp):
            r = s * CHUNKS_PER_TILE + seg * SEG
            pltpu.async_copy(src_hbm.at[pl.ds(r, SEG)], src_seg.at[sp],
                             isems[sp])
            pltpu.async_copy(dst_hbm.at[pl.ds(r, SEG)], dst_seg.at[sp],
                             isems[sp])

        def _wait_seg(seg, sp):
            r = s * CHUNKS_PER_TILE + seg * SEG
            pltpu.make_async_copy(src_hbm.at[pl.ds(r, SEG)], src_seg.at[sp],
                                  isems[sp]).wait()
            pltpu.make_async_copy(dst_hbm.at[pl.ds(r, SEG)], dst_seg.at[sp],
                                  isems[sp]).wait()

        _load_seg(0, 0)
        _load_seg(1, 1)

        plsc.subcore_barrier()

        def _gather(sp, lc, b):
            return pltpu.async_copy(table_sh.at[src_seg.at[sp, lc]],
                                    rows_v.at[b], gsems[b])

        def _wait_gather(sp, lc, b):
            pltpu.make_async_copy(table_sh.at[src_seg.at[sp, lc]],
                                  rows_v.at[b], gsems[b]).wait()

        def _scatter(sp, lc, b):
            return pltpu.async_copy(rows_v.at[b],
                                    agg_sh.at[dst_seg.at[sp, lc]],
                                    ssems[b], add=True)

        def _wait_scatter(sp, lc, b):
            pltpu.make_async_copy(rows_v.at[b],
                                  agg_sh.at[dst_seg.at[sp, lc]],
                                  ssems[b]).wait()

        def _run_segment(seg, sp):
            _wait_seg(seg, sp)
            _gather(sp, 0, 0)
            for lc in range(SEG):
                b = lc % NBUF
                nb = (lc + 1) % NBUF
                if lc + 1 < SEG:
                    if lc >= 1:
                        _wait_scatter(sp, lc - 1, nb)
                    _gather(sp, lc + 1, nb)
                _wait_gather(sp, lc, b)
                _scatter(sp, lc, b)
                if with_counts:
                    @pl.when(c == sp)
                    def _cnt():
                        pltpu.async_copy(ones_v,
                                         cnt_sh.at[dst_seg.at[sp, lc]],
                                         csem, add=True)
            _wait_scatter(sp, SEG - 2, (SEG - 2) % NBUF)
            _wait_scatter(sp, SEG - 1, (SEG - 1) % NBUF)

        def _pair(g, carry):
            for sp in range(2):
                seg = g * 2 + sp
                _run_segment(seg, sp)

                @pl.when(g < NSEG // 2 - 1)
                def _next():
                    _load_seg(seg + 2, sp)
            return carry
        lax.fori_loop(0, NSEG // 2, _pair, 0)

        if with_counts:
            def _cd(i, carry):
                pltpu.make_async_copy(ones_v, cnt_sh.at[dst_seg.at[0, 0]],
                                      csem).wait()
                return carry
            lax.fori_loop(0, CHUNKS_PER_TILE // 2, _cd, 0)

        plsc.subcore_barrier()

        pltpu.sync_copy(agg_sh.at[pl.ds(base, ROWS_PER_TILE)],
                        agg_out.at[c, pl.ds(base, ROWS_PER_TILE)])
        if with_counts:
            pltpu.sync_copy(cnt_sh.at[pl.ds(base, ROWS_PER_TILE)],
                            cnt_out.at[c, pl.ds(base, ROWS_PER_TILE)])

    mesh = plsc.VectorSubcoreMesh(core_axis_name="c", subcore_axis_name="s",
                                  num_cores=NC, num_subcores=NS)
    out_type = [jax.ShapeDtypeStruct((NC, N_PAD, width), jnp.float32)]
    scratch = [
        pltpu.VMEM((2, SEG, CHUNK), jnp.int32),
        pltpu.VMEM((2, SEG, CHUNK), jnp.int32),
        pltpu.VMEM((NBUF, CHUNK, width), jnp.float32),
    ]
    if with_counts:
        out_type.append(jax.ShapeDtypeStruct((NC, N_PAD, 16), jnp.float32))
        scratch.append(pltpu.VMEM((CHUNK, 16), jnp.float32))
        scratch.append(pltpu.VMEM((CHUNK, 16), jnp.float32))
    scratch.append(pltpu.VMEM_SHARED((N, width), jnp.float32))
    scratch.append(pltpu.VMEM_SHARED((N_PAD, width), jnp.float32))
    if with_counts:
        scratch.append(pltpu.VMEM_SHARED((N_PAD, 16), jnp.float32))
    scratch += [pltpu.SemaphoreType.DMA] * (2 * NBUF + 3)
    return pl.kernel(
        body, out_type=tuple(out_type), mesh=mesh,
        compiler_params=pltpu.CompilerParams(use_tc_tiling_on_sc=False),
        scratch_types=scratch)


_sc_agg1 = _make_sc_agg(DH, True)
_sc_agg2 = _make_sc_agg(DH, False)

BR = 2000


def _tc1_body(p_ref, cnt_ref, x_ref, wl_ref, wr_ref, b_ref, h_ref, hs_ref):
    cnt = jnp.maximum(cnt_ref[0, :, 0:1] + cnt_ref[1, :, 0:1], 1.0)
    mean = jnp.concatenate([p_ref[0], p_ref[1]], axis=1) / cnt
    h = (jnp.dot(mean, wl_ref[...], preferred_element_type=jnp.float32)
         + jnp.dot(x_ref[...], wr_ref[...], preferred_element_type=jnp.float32)
         + b_ref[...])
    h = jnp.maximum(h, 0.0)
    h_ref[...] = h
    hs_ref[0] = h[:, :DH]
    hs_ref[1] = h[:, DH:]


def _tc1(p, cnt, x, wl_t, wr_t, b):
    return pl.pallas_call(
        _tc1_body,
        grid=(N // BR,),
        in_specs=[
            pl.BlockSpec((NC, BR, DH), lambda i: (0, i, 0)),
            pl.BlockSpec((NC, BR, 16), lambda i: (0, i, 0)),
            pl.BlockSpec((BR, D), lambda i: (i, 0)),
            pl.BlockSpec((D, D), lambda i: (0, 0)),
            pl.BlockSpec((D, D), lambda i: (0, 0)),
            pl.BlockSpec((1, D), lambda i: (0, 0)),
        ],
        out_specs=[pl.BlockSpec((BR, D), lambda i: (i, 0)),
                   pl.BlockSpec((NC, BR, DH), lambda i: (0, i, 0))],
        out_shape=[jax.ShapeDtypeStruct((N, D), jnp.float32),
                   jax.ShapeDtypeStruct((NC, N, DH), jnp.float32)],
    )(p, cnt, x, wl_t, wr_t, b)


def _tc2_body(p_ref, cnt_ref, x_ref, wl_ref, wr_ref, b_ref, o_ref):
    cnt = jnp.maximum(cnt_ref[0, :, 0:1] + cnt_ref[1, :, 0:1], 1.0)
    mean = jnp.concatenate([p_ref[0], p_ref[1]], axis=1) / cnt
    o_ref[...] = (
        jnp.dot(mean, wl_ref[...], preferred_element_type=jnp.float32)
        + jnp.dot(x_ref[...], wr_ref[...], preferred_element_type=jnp.float32)
        + b_ref[...])


def _tc2(p, cnt, x, wl_t, wr_t, b):
    return pl.pallas_call(
        _tc2_body,
        grid=(N // BR,),
        in_specs=[
            pl.BlockSpec((NC, BR, DH), lambda i: (0, i, 0)),
            pl.BlockSpec((NC, BR, 16), lambda i: (0, i, 0)),
            pl.BlockSpec((BR, D), lambda i: (i, 0)),
            pl.BlockSpec((D, D), lambda i: (0, 0)),
            pl.BlockSpec((D, D), lambda i: (0, 0)),
            pl.BlockSpec((1, D), lambda i: (0, 0)),
        ],
        out_specs=pl.BlockSpec((BR, D), lambda i: (i, 0)),
        out_shape=jax.ShapeDtypeStruct((N, D), jnp.float32),
    )(p, cnt, x, wl_t, wr_t, b)


def kernel(x, edge_index, W1_l, b1_l, W1_r, W2_l, b2_l, W2_r):
    src = edge_index[0].astype(jnp.int32)
    dst = edge_index[1].astype(jnp.int32)
    pad = E_PAD - E
    src2d = jnp.concatenate(
        [src, jnp.zeros((pad,), jnp.int32)]).reshape(-1, CHUNK)
    dst2d = jnp.concatenate(
        [dst, jnp.full((pad,), N, jnp.int32)]).reshape(-1, CHUNK)

    xh = x.reshape(N, NC, DH).transpose(1, 0, 2)

    agg1, cnt = _sc_agg1(xh, src2d, dst2d)
    h, hs = _tc1(agg1, cnt, x, W1_l.T, W1_r.T, b1_l.reshape(1, D))
    (agg2,) = _sc_agg2(hs, src2d, dst2d)
    out = _tc2(agg2, cnt, h, W2_l.T, W2_r.T, b2_l.reshape(1, D))
    return out
